# baseline (speedup 1.0000x reference)
import functools

import jax
import jax.numpy as jnp
from jax.experimental import pallas as pl
from jax.experimental.pallas import tpu as pltpu

EPS = 1e-5
NUM_KEYS = 8
TOP_K = 8


def _stats_kernel(x_ref, mean_ref, inv_ref, n_total):
    i = pl.program_id(0)
    xb = x_ref[...]
    s1 = jnp.sum(xb, axis=0, keepdims=True)
    s2 = jnp.sum(xb * xb, axis=0, keepdims=True)

    @pl.when(i == 0)
    def _init():
        mean_ref[...] = s1
        inv_ref[...] = s2

    @pl.when(i > 0)
    def _acc():
        mean_ref[...] += s1
        inv_ref[...] += s2

    @pl.when(i == pl.num_programs(0) - 1)
    def _fin():
        m = mean_ref[...] / n_total
        v = inv_ref[...] / n_total - m * m
        mean_ref[...] = m
        inv_ref[...] = jax.lax.rsqrt(v + EPS)


def _main_kernel(x_ref, wg_ref, wu_ref, wd_ref, wr_ref, de_ref, ue_ref,
                 mean_ref, inv_ref, out_ref, rl_ref, acc_ref):
    j = pl.program_id(1)
    x = x_ref[...]

    @pl.when(j == 0)
    def _router_and_experts():
        inv = inv_ref[...]
        mean = mean_ref[...]
        wrs = wr_ref[...] * inv
        wrs0 = wrs[0:NUM_KEYS, :]
        wrs1 = wrs[NUM_KEYS:2 * NUM_KEYS, :]
        dn = (((1,), (1,)), ((), ()))
        rl0 = (jax.lax.dot_general(x, wrs0, dn, preferred_element_type=jnp.float32)
               - jax.lax.dot_general(mean, wrs0, dn, preferred_element_type=jnp.float32))
        rl1 = (jax.lax.dot_general(x, wrs1, dn, preferred_element_type=jnp.float32)
               - jax.lax.dot_general(mean, wrs1, dn, preferred_element_type=jnp.float32))
        rl_ref[0, :, :] = rl0
        rl_ref[1, :, :] = rl1
        m0 = jnp.max(rl0, axis=1, keepdims=True)
        lp0 = rl0 - m0 - jnp.log(jnp.sum(jnp.exp(rl0 - m0), axis=1, keepdims=True))
        m1 = jnp.max(rl1, axis=1, keepdims=True)
        lp1 = rl1 - m1 - jnp.log(jnp.sum(jnp.exp(rl1 - m1), axis=1, keepdims=True))
        ne = NUM_KEYS * NUM_KEYS
        rows = jax.lax.broadcasted_iota(jnp.int32, (NUM_KEYS, ne), 0)
        cols = jax.lax.broadcasted_iota(jnp.int32, (NUM_KEYS, ne), 1)
        sel_hi = (cols // NUM_KEYS == rows).astype(jnp.float32)
        sel_lo = (cols % NUM_KEYS == rows).astype(jnp.float32)
        dnn = (((1,), (0,)), ((), ()))
        scores = (jax.lax.dot_general(lp0, sel_hi, dnn, preferred_element_type=jnp.float32)
                  + jax.lax.dot_general(lp1, sel_lo, dnn, preferred_element_type=jnp.float32))
        cur = scores
        for _ in range(TOP_K - 1):
            mx = jnp.max(cur, axis=1, keepdims=True)
            cur = jnp.where(cur >= mx, -jnp.inf, cur)
        thr = jnp.max(cur, axis=1, keepdims=True)
        w = jnp.where(scores >= thr, jnp.exp(scores), 0.0)
        h = jax.lax.dot_general(x, de_ref[...], dn, preferred_element_type=jnp.float32)
        g = h * jax.nn.sigmoid(h) * w
        acc_ref[...] = jax.lax.dot_general(g, ue_ref[...], dnn,
                                           preferred_element_type=jnp.float32)

    dn = (((1,), (1,)), ((), ()))
    xb = x.astype(jnp.bfloat16)
    gate = jax.lax.dot_general(xb, wg_ref[...], dn, preferred_element_type=jnp.float32)
    up = jax.lax.dot_general(xb, wu_ref[...], dn, preferred_element_type=jnp.float32)
    a = (gate * jax.nn.sigmoid(gate) * up).astype(jnp.bfloat16)
    acc_ref[...] += jax.lax.dot_general(a, wd_ref[...], dn,
                                        preferred_element_type=jnp.float32)

    @pl.when(j == pl.num_programs(1) - 1)
    def _emit():
        out_ref[...] = acc_ref[...]


def kernel(hidden_states, W_gate, W_up, W_down, W_router, down_embed, up_embed):
    bsz, seq_len, hidden = hidden_states.shape
    inter = W_gate.shape[0]
    n = bsz * seq_len
    x = hidden_states.reshape(n, hidden)

    rb = min(1024, n)
    mean, inv = pl.pallas_call(
        functools.partial(_stats_kernel, n_total=float(n)),
        grid=(n // rb,),
        in_specs=[pl.BlockSpec((rb, hidden), lambda i: (i, 0))],
        out_specs=[pl.BlockSpec((1, hidden), lambda i: (0, 0)),
                   pl.BlockSpec((1, hidden), lambda i: (0, 0))],
        out_shape=[jax.ShapeDtypeStruct((1, hidden), jnp.float32),
                   jax.ShapeDtypeStruct((1, hidden), jnp.float32)],
        compiler_params=pltpu.CompilerParams(
            dimension_semantics=("arbitrary",)),
    )(x)

    tb = min(1024, n)
    ic = min(512, inter)
    grid = (n // tb, inter // ic)
    out, rl = pl.pallas_call(
        _main_kernel,
        grid=grid,
        in_specs=[
            pl.BlockSpec((tb, hidden), lambda t, j: (t, 0)),
            pl.BlockSpec((ic, hidden), lambda t, j: (j, 0)),
            pl.BlockSpec((ic, hidden), lambda t, j: (j, 0)),
            pl.BlockSpec((hidden, ic), lambda t, j: (0, j)),
            pl.BlockSpec((2 * NUM_KEYS, hidden), lambda t, j: (0, 0)),
            pl.BlockSpec((NUM_KEYS * NUM_KEYS, hidden), lambda t, j: (0, 0)),
            pl.BlockSpec((NUM_KEYS * NUM_KEYS, hidden), lambda t, j: (0, 0)),
            pl.BlockSpec((1, hidden), lambda t, j: (0, 0)),
            pl.BlockSpec((1, hidden), lambda t, j: (0, 0)),
        ],
        out_specs=[
            pl.BlockSpec((tb, hidden), lambda t, j: (t, 0)),
            pl.BlockSpec((2, tb, NUM_KEYS), lambda t, j: (0, t, 0)),
        ],
        out_shape=[
            jax.ShapeDtypeStruct((n, hidden), jnp.float32),
            jax.ShapeDtypeStruct((2, n, NUM_KEYS), jnp.float32),
        ],
        scratch_shapes=[pltpu.VMEM((tb, hidden), jnp.float32)],
        compiler_params=pltpu.CompilerParams(
            dimension_semantics=("parallel", "arbitrary")),
    )(x, W_gate.astype(jnp.bfloat16), W_up.astype(jnp.bfloat16),
      W_down.astype(jnp.bfloat16), W_router, down_embed, up_embed, mean, inv)

    return (out.reshape(bsz, seq_len, hidden), rl)


# speedup vs baseline: 1.1347x; 1.1347x over previous
import functools

import jax
import jax.numpy as jnp
from jax.experimental import pallas as pl
from jax.experimental.pallas import tpu as pltpu

EPS = 1e-5
NUM_KEYS = 8
TOP_K = 8


def _stats_kernel(x_ref, mean_ref, inv_ref, n_total):
    i = pl.program_id(0)
    xb = x_ref[...]
    s1 = jnp.sum(xb, axis=0, keepdims=True)
    s2 = jnp.sum(xb * xb, axis=0, keepdims=True)

    @pl.when(i == 0)
    def _init():
        mean_ref[...] = s1
        inv_ref[...] = s2

    @pl.when(i > 0)
    def _acc():
        mean_ref[...] += s1
        inv_ref[...] += s2

    @pl.when(i == pl.num_programs(0) - 1)
    def _fin():
        m = mean_ref[...] / n_total
        v = inv_ref[...] / n_total - m * m
        mean_ref[...] = m
        inv_ref[...] = jax.lax.rsqrt(v + EPS)


def _main_kernel(x_ref, wg_ref, wu_ref, wd_ref, wr_ref, de_ref, ue_ref,
                 mean_ref, inv_ref, out_ref, rl_ref, acc_ref):
    j = pl.program_id(1)
    x = x_ref[...]

    @pl.when(j == 0)
    def _router_and_experts():
        inv = inv_ref[...]
        mean = mean_ref[...]
        wrs = wr_ref[...] * inv
        wrs0 = wrs[0:NUM_KEYS, :]
        wrs1 = wrs[NUM_KEYS:2 * NUM_KEYS, :]
        dn = (((1,), (1,)), ((), ()))
        rl0 = (jax.lax.dot_general(x, wrs0, dn, preferred_element_type=jnp.float32)
               - jax.lax.dot_general(mean, wrs0, dn, preferred_element_type=jnp.float32))
        rl1 = (jax.lax.dot_general(x, wrs1, dn, preferred_element_type=jnp.float32)
               - jax.lax.dot_general(mean, wrs1, dn, preferred_element_type=jnp.float32))
        rl_ref[0, :, :] = rl0
        rl_ref[1, :, :] = rl1
        m0 = jnp.max(rl0, axis=1, keepdims=True)
        lp0 = rl0 - m0 - jnp.log(jnp.sum(jnp.exp(rl0 - m0), axis=1, keepdims=True))
        m1 = jnp.max(rl1, axis=1, keepdims=True)
        lp1 = rl1 - m1 - jnp.log(jnp.sum(jnp.exp(rl1 - m1), axis=1, keepdims=True))
        ne = NUM_KEYS * NUM_KEYS
        rows = jax.lax.broadcasted_iota(jnp.int32, (NUM_KEYS, ne), 0)
        cols = jax.lax.broadcasted_iota(jnp.int32, (NUM_KEYS, ne), 1)
        sel_hi = (cols // NUM_KEYS == rows).astype(jnp.float32)
        sel_lo = (cols % NUM_KEYS == rows).astype(jnp.float32)
        dnn = (((1,), (0,)), ((), ()))
        scores = (jax.lax.dot_general(lp0, sel_hi, dnn, preferred_element_type=jnp.float32)
                  + jax.lax.dot_general(lp1, sel_lo, dnn, preferred_element_type=jnp.float32))
        cur = scores
        for _ in range(TOP_K - 1):
            mx = jnp.max(cur, axis=1, keepdims=True)
            cur = jnp.where(cur >= mx, -jnp.inf, cur)
        thr = jnp.max(cur, axis=1, keepdims=True)
        w = jnp.where(scores >= thr, jnp.exp(scores), 0.0)
        h = jax.lax.dot_general(x, de_ref[...], dn, preferred_element_type=jnp.float32)
        g = h * jax.nn.sigmoid(h) * w
        acc_ref[...] = jax.lax.dot_general(g, ue_ref[...], dnn,
                                           preferred_element_type=jnp.float32)

    dn = (((1,), (1,)), ((), ()))
    gate = jax.lax.dot_general(x, wg_ref[...], dn, preferred_element_type=jnp.float32)
    up = jax.lax.dot_general(x, wu_ref[...], dn, preferred_element_type=jnp.float32)
    a = gate * jax.nn.sigmoid(gate) * up
    acc_ref[...] += jax.lax.dot_general(a, wd_ref[...], dn,
                                        preferred_element_type=jnp.float32)

    @pl.when(j == pl.num_programs(1) - 1)
    def _emit():
        out_ref[...] = acc_ref[...]


def kernel(hidden_states, W_gate, W_up, W_down, W_router, down_embed, up_embed):
    bsz, seq_len, hidden = hidden_states.shape
    inter = W_gate.shape[0]
    n = bsz * seq_len
    x = hidden_states.reshape(n, hidden)

    rb = min(1024, n)
    mean, inv = pl.pallas_call(
        functools.partial(_stats_kernel, n_total=float(n)),
        grid=(n // rb,),
        in_specs=[pl.BlockSpec((rb, hidden), lambda i: (i, 0))],
        out_specs=[pl.BlockSpec((1, hidden), lambda i: (0, 0)),
                   pl.BlockSpec((1, hidden), lambda i: (0, 0))],
        out_shape=[jax.ShapeDtypeStruct((1, hidden), jnp.float32),
                   jax.ShapeDtypeStruct((1, hidden), jnp.float32)],
        compiler_params=pltpu.CompilerParams(
            dimension_semantics=("arbitrary",)),
    )(x)

    tb = min(1024, n)
    ic = min(1024, inter)
    grid = (n // tb, inter // ic)
    out, rl = pl.pallas_call(
        _main_kernel,
        grid=grid,
        in_specs=[
            pl.BlockSpec((tb, hidden), lambda t, j: (t, 0)),
            pl.BlockSpec((ic, hidden), lambda t, j: (j, 0)),
            pl.BlockSpec((ic, hidden), lambda t, j: (j, 0)),
            pl.BlockSpec((hidden, ic), lambda t, j: (0, j)),
            pl.BlockSpec((2 * NUM_KEYS, hidden), lambda t, j: (0, 0)),
            pl.BlockSpec((NUM_KEYS * NUM_KEYS, hidden), lambda t, j: (0, 0)),
            pl.BlockSpec((NUM_KEYS * NUM_KEYS, hidden), lambda t, j: (0, 0)),
            pl.BlockSpec((1, hidden), lambda t, j: (0, 0)),
            pl.BlockSpec((1, hidden), lambda t, j: (0, 0)),
        ],
        out_specs=[
            pl.BlockSpec((tb, hidden), lambda t, j: (t, 0)),
            pl.BlockSpec((2, tb, NUM_KEYS), lambda t, j: (0, t, 0)),
        ],
        out_shape=[
            jax.ShapeDtypeStruct((n, hidden), jnp.float32),
            jax.ShapeDtypeStruct((2, n, NUM_KEYS), jnp.float32),
        ],
        scratch_shapes=[pltpu.VMEM((tb, hidden), jnp.float32)],
        compiler_params=pltpu.CompilerParams(
            dimension_semantics=("parallel", "arbitrary")),
    )(x, W_gate, W_up, W_down, W_router, down_embed, up_embed, mean, inv)

    return (out.reshape(bsz, seq_len, hidden), rl)
